# baseline (speedup 1.0000x reference)
import functools

import jax
import jax.numpy as jnp
from jax import lax
from jax.experimental import pallas as pl
from jax.experimental.pallas import tpu as pltpu
from jax.experimental.pallas import tpu_sc as plsc

B = 16384
F = 32
L = 16
NC = 2
NS = 16
NW = NC * NS
BPW = B // NW
GROUPS = BPW // L

_mesh = plsc.VectorSubcoreMesh(core_axis_name="c", subcore_axis_name="s")


@functools.partial(
    pl.kernel,
    mesh=_mesh,
    out_type=jax.ShapeDtypeStruct((B,), jnp.float32),
    scratch_types=[
        pltpu.VMEM((BPW,), jnp.int32),
        pltpu.VMEM((BPW,), jnp.int32),
        pltpu.VMEM((BPW, F), jnp.float32),
        pltpu.VMEM((BPW, F), jnp.float32),
        pltpu.VMEM((BPW,), jnp.float32),
        pltpu.SemaphoreType.DMA,
        pltpu.SemaphoreType.DMA,
    ],
    compiler_params=pltpu.CompilerParams(
        use_tc_tiling_on_sc=False, needs_layout_passes=False),
)
def _mf_score(users_hbm, items_hbm, uw_hbm, iw_hbm, out_hbm,
              uidx_v, iidx_v, urows_v, irows_v, out_v, usem, vsem):
    wid = lax.axis_index("s") * NC + lax.axis_index("c")
    base = wid * BPW

    pltpu.sync_copy(users_hbm.at[pl.ds(base, BPW)], uidx_v)
    pltpu.sync_copy(items_hbm.at[pl.ds(base, BPW)], iidx_v)

    cu = pltpu.async_copy(uw_hbm.at[uidx_v], urows_v, usem)
    cv = pltpu.async_copy(iw_hbm.at[iidx_v], irows_v, vsem)
    cu.wait()
    cv.wait()

    lanes = lax.iota(jnp.int32, L)

    def group(g, carry):
        rows = g * L + lanes
        acc0 = jnp.zeros((L,), jnp.float32)
        acc1 = jnp.zeros((L,), jnp.float32)
        for f in range(F):
            col = jnp.full((L,), f, jnp.int32)
            uv = plsc.load_gather(urows_v, [rows, col])
            vv = plsc.load_gather(irows_v, [rows, col])
            if f % 2 == 0:
                acc0 = acc0 + uv * vv
            else:
                acc1 = acc1 + uv * vv
        out_v[pl.ds(g * L, L)] = acc0 + acc1
        return carry

    lax.fori_loop(0, GROUPS, group, 0)

    pltpu.sync_copy(out_v, out_hbm.at[pl.ds(base, BPW)])


def kernel(users, items, user_weight, item_weight):
    return _mf_score(users.astype(jnp.int32), items.astype(jnp.int32),
                     user_weight, item_weight)


# speedup vs baseline: 3.8683x; 3.8683x over previous
import functools

import jax
import jax.numpy as jnp
from jax import lax
from jax.experimental import pallas as pl
from jax.experimental.pallas import tpu as pltpu
from jax.experimental.pallas import tpu_sc as plsc

B = 16384
F = 32
L = 16
NC = 2
NS = 16
NW = NC * NS
BPW = B // NW
G = 4
NWAVE = BPW // G

_mesh = plsc.VectorSubcoreMesh(core_axis_name="c", subcore_axis_name="s")


@functools.partial(
    pl.kernel,
    mesh=_mesh,
    out_type=jax.ShapeDtypeStruct((B,), jnp.float32),
    scratch_types=[
        pltpu.VMEM((BPW,), jnp.int32),
        pltpu.VMEM((BPW,), jnp.int32),
        pltpu.VMEM((2, G, F, 128), jnp.float32),
        pltpu.VMEM((2, G, F, 128), jnp.float32),
        pltpu.VMEM((BPW,), jnp.float32),
        pltpu.SemaphoreType.DMA((2,)),
        pltpu.SemaphoreType.DMA((2,)),
    ],
    compiler_params=pltpu.CompilerParams(needs_layout_passes=False),
)
def _mf_score(users_hbm, items_hbm, uwt_hbm, iwt_hbm, out_hbm,
              uidx_v, iidx_v, ubuf_v, ibuf_v, out_v, usem, vsem):
    wid = lax.axis_index("s") * NC + lax.axis_index("c")
    base = wid * BPW

    pltpu.sync_copy(users_hbm.at[pl.ds(base, BPW)], uidx_v)
    pltpu.sync_copy(items_hbm.at[pl.ds(base, BPW)], iidx_v)

    lanes = lax.iota(jnp.int32, L)

    def scalar_at(vec, lane):
        return jnp.sum(jnp.where(lanes == lane, vec, 0))

    def fire_wave(w, slot):
        uvec = uidx_v[pl.ds((w // G) * L, L)]
        ivec = iidx_v[pl.ds((w // G) * L, L)]
        for jj in range(G):
            lane = (w % G) * G + jj
            u = scalar_at(uvec, lane)
            i = scalar_at(ivec, lane)
            ug = pl.multiple_of(
                lax.shift_right_logical(u, 7) * 128, 128)
            ig = pl.multiple_of(
                lax.shift_right_logical(i, 7) * 128, 128)
            pltpu.async_copy(uwt_hbm.at[:, pl.ds(ug, 128)],
                             ubuf_v.at[slot, jj], usem.at[slot])
            pltpu.async_copy(iwt_hbm.at[:, pl.ds(ig, 128)],
                             ibuf_v.at[slot, jj], vsem.at[slot])

    def drain_wave(slot):
        pltpu.make_async_copy(
            uwt_hbm.at[:, pl.ds(0, G * 128)],
            ubuf_v.at[slot].reshape(G * F, 128), usem.at[slot]).wait()
        pltpu.make_async_copy(
            iwt_hbm.at[:, pl.ds(0, G * 128)],
            ibuf_v.at[slot].reshape(G * F, 128), vsem.at[slot]).wait()

    fire_wave(0, 0)

    def wave(w, acc):
        slot = w & 1

        @pl.when(w + 1 < NWAVE)
        def _():
            fire_wave(w + 1, 1 - slot)

        drain_wave(slot)

        uvec = uidx_v[pl.ds((w // G) * L, L)]
        ivec = iidx_v[pl.ds((w // G) * L, L)]
        for jj in range(G):
            j = w * G + jj
            lane = (w % G) * G + jj
            uc = scalar_at(uvec, lane) & 127
            ic = scalar_at(ivec, lane) & 127
            u0 = plsc.load_gather(ubuf_v.at[slot, jj],
                                  [lanes, jnp.full((L,), uc, jnp.int32)])
            u1 = plsc.load_gather(ubuf_v.at[slot, jj],
                                  [lanes + L, jnp.full((L,), uc, jnp.int32)])
            v0 = plsc.load_gather(ibuf_v.at[slot, jj],
                                  [lanes, jnp.full((L,), ic, jnp.int32)])
            v1 = plsc.load_gather(ibuf_v.at[slot, jj],
                                  [lanes + L, jnp.full((L,), ic, jnp.int32)])
            s = jnp.sum(u0 * v0 + u1 * v1)
            sel = lanes == (j % L)
            acc = jnp.where(sel, s, acc)

        @pl.when((w & (L // G - 1)) == (L // G - 1))
        def _():
            out_v[pl.ds((w // (L // G)) * L, L)] = acc

        return acc

    lax.fori_loop(0, NWAVE, wave, jnp.zeros((L,), jnp.float32))

    pltpu.sync_copy(out_v, out_hbm.at[pl.ds(base, BPW)])


def kernel(users, items, user_weight, item_weight):
    return _mf_score(users.astype(jnp.int32), items.astype(jnp.int32),
                     user_weight.T, item_weight.T)


# speedup vs baseline: 4.3001x; 1.1116x over previous
import functools

import jax
import jax.numpy as jnp
from jax import lax
from jax.experimental import pallas as pl
from jax.experimental.pallas import tpu as pltpu
from jax.experimental.pallas import tpu_sc as plsc

B = 16384
F = 32
L = 16
NC = 2
NS = 16
NW = NC * NS
BPW = B // NW
G = 4
NWAVE = BPW // G
SLOTS = 3

_mesh = plsc.VectorSubcoreMesh(core_axis_name="c", subcore_axis_name="s")


@functools.partial(
    pl.kernel,
    mesh=_mesh,
    out_type=jax.ShapeDtypeStruct((B,), jnp.float32),
    scratch_types=[
        pltpu.VMEM((BPW,), jnp.int32),
        pltpu.VMEM((BPW,), jnp.int32),
        pltpu.VMEM((SLOTS, G, F, 128), jnp.float32),
        pltpu.VMEM((SLOTS, G, F, 128), jnp.float32),
        pltpu.VMEM((BPW,), jnp.float32),
        pltpu.SemaphoreType.DMA((SLOTS,)),
        pltpu.SemaphoreType.DMA((SLOTS,)),
    ],
    compiler_params=pltpu.CompilerParams(needs_layout_passes=False),
)
def _mf_score(users_hbm, items_hbm, uwt_hbm, iwt_hbm, out_hbm,
              uidx_v, iidx_v, ubuf_v, ibuf_v, out_v, usem, vsem):
    wid = lax.axis_index("s") * NC + lax.axis_index("c")
    base = wid * BPW

    pltpu.sync_copy(users_hbm.at[pl.ds(base, BPW)], uidx_v)
    pltpu.sync_copy(items_hbm.at[pl.ds(base, BPW)], iidx_v)

    lanes = lax.iota(jnp.int32, L)

    def scalar_at(vec, lane):
        return jnp.sum(jnp.where(lanes == lane, vec, 0))

    def fire_wave(w, slot):
        uvec = uidx_v[pl.ds((w // G) * L, L)]
        ivec = iidx_v[pl.ds((w // G) * L, L)]
        for jj in range(G):
            lane = (w % G) * G + jj
            u = scalar_at(uvec, lane)
            i = scalar_at(ivec, lane)
            ug = pl.multiple_of(
                lax.shift_right_logical(u, 7) * 128, 128)
            ig = pl.multiple_of(
                lax.shift_right_logical(i, 7) * 128, 128)
            pltpu.async_copy(uwt_hbm.at[:, pl.ds(ug, 128)],
                             ubuf_v.at[slot, jj], usem.at[slot])
            pltpu.async_copy(iwt_hbm.at[:, pl.ds(ig, 128)],
                             ibuf_v.at[slot, jj], vsem.at[slot])

    def drain_wave(slot):
        pltpu.make_async_copy(
            uwt_hbm.at[:, pl.ds(0, G * 128)],
            ubuf_v.at[slot].reshape(G * F, 128), usem.at[slot]).wait()
        pltpu.make_async_copy(
            iwt_hbm.at[:, pl.ds(0, G * 128)],
            ibuf_v.at[slot].reshape(G * F, 128), vsem.at[slot]).wait()

    fire_wave(0, 0)
    fire_wave(1, 1)

    def wave(w, acc):
        slot = lax.rem(w, SLOTS)

        @pl.when(w + 2 < NWAVE)
        def _():
            fire_wave(w + 2, lax.rem(w + 2, SLOTS))

        drain_wave(slot)

        uvec = uidx_v[pl.ds((w // G) * L, L)]
        ivec = iidx_v[pl.ds((w // G) * L, L)]
        for jj in range(G):
            j = w * G + jj
            lane = (w % G) * G + jj
            uc = scalar_at(uvec, lane) & 127
            ic = scalar_at(ivec, lane) & 127
            u0 = plsc.load_gather(ubuf_v.at[slot, jj],
                                  [lanes, jnp.full((L,), uc, jnp.int32)])
            u1 = plsc.load_gather(ubuf_v.at[slot, jj],
                                  [lanes + L, jnp.full((L,), uc, jnp.int32)])
            v0 = plsc.load_gather(ibuf_v.at[slot, jj],
                                  [lanes, jnp.full((L,), ic, jnp.int32)])
            v1 = plsc.load_gather(ibuf_v.at[slot, jj],
                                  [lanes + L, jnp.full((L,), ic, jnp.int32)])
            s = jnp.sum(u0 * v0 + u1 * v1)
            sel = lanes == (j % L)
            acc = jnp.where(sel, s, acc)

        @pl.when((w & (L // G - 1)) == (L // G - 1))
        def _():
            out_v[pl.ds((w // (L // G)) * L, L)] = acc

        return acc

    lax.fori_loop(0, NWAVE, wave, jnp.zeros((L,), jnp.float32))

    pltpu.sync_copy(out_v, out_hbm.at[pl.ds(base, BPW)])


def kernel(users, items, user_weight, item_weight):
    return _mf_score(users.astype(jnp.int32), items.astype(jnp.int32),
                     user_weight.T, item_weight.T)


# speedup vs baseline: 4.4713x; 1.0398x over previous
import functools

import jax
import jax.numpy as jnp
from jax import lax
from jax.experimental import pallas as pl
from jax.experimental.pallas import tpu as pltpu
from jax.experimental.pallas import tpu_sc as plsc

B = 16384
F = 32
L = 16
NC = 2
NS = 16
NW = NC * NS
BPW = B // NW
G = 4
NWAVE = BPW // G
SLOTS = 3

_mesh = plsc.VectorSubcoreMesh(core_axis_name="c", subcore_axis_name="s")


@functools.partial(
    pl.kernel,
    mesh=_mesh,
    out_type=jax.ShapeDtypeStruct((B,), jnp.float32),
    scratch_types=[
        pltpu.VMEM((BPW,), jnp.int32),
        pltpu.VMEM((BPW,), jnp.int32),
        pltpu.VMEM((SLOTS, G, F, 128), jnp.float32),
        pltpu.VMEM((SLOTS, G, F, 128), jnp.float32),
        pltpu.VMEM((BPW,), jnp.float32),
        pltpu.SemaphoreType.DMA((SLOTS,)),
        pltpu.SemaphoreType.DMA((SLOTS,)),
    ],
    compiler_params=pltpu.CompilerParams(needs_layout_passes=False),
)
def _mf_score(users_hbm, items_hbm, uwt_hbm, iwt_hbm, out_hbm,
              uidx_v, iidx_v, ubuf_v, ibuf_v, out_v, usem, vsem):
    wid = lax.axis_index("s") * NC + lax.axis_index("c")
    base = wid * BPW

    pltpu.sync_copy(users_hbm.at[pl.ds(base, BPW)], uidx_v)
    pltpu.sync_copy(items_hbm.at[pl.ds(base, BPW)], iidx_v)

    lanes = lax.iota(jnp.int32, L)

    def scalar_at(vec, lane):
        return jnp.sum(jnp.where(lanes == lane, vec, 0))

    def wave_groups(w):
        uvec = uidx_v[pl.ds((w // G) * L, L)]
        ugs = []
        for jj in range(G):
            lane = (w % G) * G + jj
            ugs.append(lax.shift_right_logical(scalar_at(uvec, lane), 7))
        news = [None] + [ugs[jj] != ugs[jj - 1] for jj in range(1, G)]
        return ugs, news

    def fire_wave(w, slot):
        ivec = iidx_v[pl.ds((w // G) * L, L)]
        ugs, news = wave_groups(w)
        for jj in range(G):
            lane = (w % G) * G + jj
            ug = pl.multiple_of(ugs[jj] * 128, 128)
            ig = pl.multiple_of(
                lax.shift_right_logical(scalar_at(ivec, lane), 7) * 128, 128)

            def fire_u(ug=ug, jj=jj):
                pltpu.async_copy(uwt_hbm.at[:, pl.ds(ug, 128)],
                                 ubuf_v.at[slot, jj], usem.at[slot])

            if jj == 0:
                fire_u()
            else:
                pl.when(news[jj])(fire_u)
            pltpu.async_copy(iwt_hbm.at[:, pl.ds(ig, 128)],
                             ibuf_v.at[slot, jj], vsem.at[slot])

    def drain_wave(w, slot):
        _, news = wave_groups(w)

        def drain_u(slot=slot):
            pltpu.make_async_copy(
                uwt_hbm.at[:, pl.ds(0, 128)],
                ubuf_v.at[slot, 0], usem.at[slot]).wait()

        drain_u()
        for jj in range(1, G):
            pl.when(news[jj])(drain_u)
        pltpu.make_async_copy(
            iwt_hbm.at[:, pl.ds(0, G * 128)],
            ibuf_v.at[slot].reshape(G * F, 128), vsem.at[slot]).wait()

    fire_wave(0, 0)
    fire_wave(1, 1)

    def wave(w, acc):
        slot = lax.rem(w, SLOTS)

        @pl.when(w + 2 < NWAVE)
        def _():
            fire_wave(w + 2, lax.rem(w + 2, SLOTS))

        drain_wave(w, slot)

        uvec = uidx_v[pl.ds((w // G) * L, L)]
        ivec = iidx_v[pl.ds((w // G) * L, L)]
        _, news = wave_groups(w)
        src = jnp.int32(0)
        for jj in range(G):
            j = w * G + jj
            lane = (w % G) * G + jj
            if jj > 0:
                src = jnp.where(news[jj], jj, src)
            uc = scalar_at(uvec, lane) & 127
            ic = scalar_at(ivec, lane) & 127
            u0 = plsc.load_gather(ubuf_v.at[slot, src],
                                  [lanes, jnp.full((L,), uc, jnp.int32)])
            u1 = plsc.load_gather(ubuf_v.at[slot, src],
                                  [lanes + L, jnp.full((L,), uc, jnp.int32)])
            v0 = plsc.load_gather(ibuf_v.at[slot, jj],
                                  [lanes, jnp.full((L,), ic, jnp.int32)])
            v1 = plsc.load_gather(ibuf_v.at[slot, jj],
                                  [lanes + L, jnp.full((L,), ic, jnp.int32)])
            s = jnp.sum(u0 * v0 + u1 * v1)
            sel = lanes == (j % L)
            acc = jnp.where(sel, s, acc)

        @pl.when((w & (L // G - 1)) == (L // G - 1))
        def _():
            out_v[pl.ds((w // (L // G)) * L, L)] = acc

        return acc

    lax.fori_loop(0, NWAVE, wave, jnp.zeros((L,), jnp.float32))

    pltpu.sync_copy(out_v, out_hbm.at[pl.ds(base, BPW)])


def kernel(users, items, user_weight, item_weight):
    users = users.astype(jnp.int32)
    items = items.astype(jnp.int32)
    order = jnp.argsort(users)
    res = _mf_score(jnp.take(users, order), jnp.take(items, order),
                    user_weight.T, item_weight.T)
    inv = jnp.zeros_like(order).at[order].set(
        jnp.arange(B, dtype=order.dtype))
    return jnp.take(res, inv)
